# baseline (speedup 1.0000x reference)
import functools
import numpy as np
import jax
import jax.numpy as jnp
from jax import lax
from jax.experimental import pallas as pl
from jax.experimental.pallas import tpu as pltpu
from jax.experimental.pallas import tpu_sc as plsc

N = 20000
K = 1000
L = 16
NC = 2
NT = 16
NPAD = 20480
SH = NPAD // NT
SV = SH // L
CAP = 1024
SLOTS = CAP // (NC * NT)
GROUPS = SLOTS // L

NR = 32
A_CGT = NR
A_EVEC = NR + 1
A_CVEC = NR + 17
A_TOT = NR + 33

_MSK = np.int32(0x7FFFFFFF)
_ONE = np.int32(1)
_ZERO = np.int32(0)
_IMIN = np.int32(-0x80000000)


def _to_key(s):
    b = lax.bitcast_convert_type(s, jnp.int32)
    return jnp.where(b < 0, b ^ _MSK, b)


def _from_key(k):
    b = jnp.where(k < 0, k ^ _MSK, k)
    return lax.bitcast_convert_type(b, jnp.float32)


def _sc_body(scores_hbm, boxes_flat_hbm, out_hbm, xk_hbm, xi_hbm,
             sv, kv, ck, ci, rk, ri, mk, mi, colb, stage5, smem, sem):
    core = lax.axis_index("c")
    w = lax.axis_index("s")
    ii = lax.iota(jnp.int32, L)

    def _z(i, _):
        smem[i] = _ZERO
        return 0
    lax.fori_loop(0, A_TOT, _z, 0)

    pltpu.sync_copy(scores_hbm.at[pl.ds(w * SH, SH)], sv)
    for i in range(SV):
        kv[pl.ds(i * L, L)] = _to_key(sv[pl.ds(i * L, L)])

    plsc.subcore_barrier()

    def _count(pred):
        acc = jnp.zeros((L,), jnp.int32)
        for i in range(SV):
            v = kv[pl.ds(i * L, L)]
            acc = acc + jnp.where(pred(v), _ONE, _ZERO)
        return jnp.sum(acc)

    def _acc_read(word, local):
        plsc.fetch_and_add(smem.at[word], local, subcore_id=0)
        plsc.subcore_barrier()
        return plsc.fetch_and_add(smem.at[word], _ZERO, subcore_id=0)

    def _round(r, bstate):
        cand_b = bstate | (_ONE << (31 - r))
        cand = cand_b ^ _IMIN
        total = _acc_read(r, _count(lambda v: v >= cand))
        return jnp.where(total >= K, cand_b, bstate)

    t_key = lax.fori_loop(0, NR, _round, _ZERO) ^ _IMIN

    c_gt = _acc_read(A_CGT, _count(lambda v: v > t_key))
    need_eq = np.int32(K) - c_gt

    e_local = _count(lambda v: v == t_key)
    plsc.fetch_and_add(smem.at[A_EVEC + w], e_local, subcore_id=0)
    plsc.subcore_barrier()

    def _pref(j, acc):
        return acc + plsc.fetch_and_add(smem.at[A_EVEC + j], _ZERO,
                                        subcore_id=0)
    e_before = lax.fori_loop(0, w, _pref, _ZERO)
    take_eq = jnp.clip(need_eq - e_before, _ZERO, e_local)

    def _comp(i, carry):
        off, eqs = carry
        v = kv[pl.ds(i * L, L)]
        m_gt = v > t_key
        m_eq = v == t_key
        eq_in = jnp.where(m_eq, _ONE, _ZERO)
        eq_rank = plsc.cumsum(eq_in) - eq_in + eqs
        sel = m_gt | (m_eq & (eq_rank < take_eq))
        idx = w * SH + i * L + ii
        sel_in = jnp.where(sel, _ONE, _ZERO)
        pos = off + plsc.cumsum(sel_in) - sel_in
        plsc.store_scatter(ck, [pos], v, mask=sel)
        plsc.store_scatter(ci, [pos], idx, mask=sel)
        return off + jnp.sum(sel_in), eqs + jnp.sum(eq_in)

    c_local, _ = lax.fori_loop(0, SV, _comp, (_ZERO, _ZERO))

    row = core * NT + w
    pltpu.sync_copy(ck.at[pl.ds(0, CAP)], xk_hbm.at[row])
    pltpu.sync_copy(ci.at[pl.ds(0, CAP)], xi_hbm.at[row])
    plsc.fetch_and_add(smem.at[A_CVEC + w], c_local, subcore_id=0)
    plsc.subcore_barrier()

    pltpu.sync_copy(xk_hbm.at[pl.ds(core * NT, NT)], rk)
    pltpu.sync_copy(xi_hbm.at[pl.ds(core * NT, NT)], ri)

    def _cnt_of(j):
        return plsc.fetch_and_add(smem.at[A_CVEC + j], _ZERO, subcore_id=0)

    pos_run = _ZERO
    for j in range(NT):
        cj = _cnt_of(j)
        pj = pos_run

        def _mrg(t, _, j=j, cj=cj, pj=pj):
            vv = rk[j, pl.ds(t * L, L)]
            iv = ri[j, pl.ds(t * L, L)]
            lpos = t * L + ii
            mm = lpos < cj
            plsc.store_scatter(mk, [pj + lpos], vv, mask=mm)
            plsc.store_scatter(mi, [pj + lpos], iv, mask=mm)
            return 0

        lax.fori_loop(0, (cj + (L - 1)) // L, _mrg, 0)
        pos_run = pos_run + cj

    for tpos in range((K // L), CAP // L):
        base = tpos * L
        gp = base + ii
        old_k = mk[pl.ds(base, L)]
        old_i = mi[pl.ds(base, L)]
        pad = gp >= K
        mk[pl.ds(base, L)] = jnp.where(pad, _IMIN, old_k)
        mi[pl.ds(base, L)] = jnp.where(pad, np.int32(N) + gp, old_i)

    for g in range(GROUPS):
        base = (core * NT + w) * SLOTS + g * L
        sk = mk[pl.ds(base, L)]
        si = mi[pl.ds(base, L)]

        def _rank(t, acc, sk=sk, si=si):
            ek = mk[pl.ds(t * L, L)]
            ei = mi[pl.ds(t * L, L)]
            for p in range(L):
                perm = (ii + p) & (L - 1)
                ekp = ek.at[perm].get(mode="promise_in_bounds")
                eip = ei.at[perm].get(mode="promise_in_bounds")
                beats = (ekp > sk) | ((ekp == sk) & (eip < si))
                acc = acc + jnp.where(beats, _ONE, _ZERO)
            return acc

        rank = lax.fori_loop(0, CAP // L, _rank, jnp.zeros((L,), jnp.int32))

        gidx = jnp.where(si < N, si, si - np.int32(N))
        copies = []
        for c in range(4):
            copies.append(
                pltpu.async_copy(boxes_flat_hbm.at[gidx * 4 + c],
                                 colb.at[c], sem))
        stage5[...] = _from_key(sk)
        for c in range(4):
            copies[c].wait()
        outp = []
        for c in range(4):
            outp.append(
                pltpu.async_copy(colb.at[c], out_hbm.at[rank * 5 + c], sem))
        outp.append(
            pltpu.async_copy(stage5, out_hbm.at[rank * 5 + 4], sem))
        for d in outp:
            d.wait()


@jax.jit
def kernel(boxes, scores):
    scores_pad = jnp.concatenate(
        [scores, jnp.full((NPAD - N,), -jnp.inf, jnp.float32)])
    boxes_flat = boxes.reshape(-1)

    mesh = plsc.VectorSubcoreMesh(
        core_axis_name="c", subcore_axis_name="s",
        num_cores=NC, num_subcores=NT)
    out, _, _ = pl.kernel(
        _sc_body,
        out_type=(
            jax.ShapeDtypeStruct((CAP * 5,), jnp.float32),
            jax.ShapeDtypeStruct((NC * NT, CAP), jnp.int32),
            jax.ShapeDtypeStruct((NC * NT, CAP), jnp.int32),
        ),
        mesh=mesh,
        compiler_params=pltpu.CompilerParams(needs_layout_passes=False),
        scratch_types=[
            pltpu.VMEM((SH,), jnp.float32),
            pltpu.VMEM((SH,), jnp.int32),
            pltpu.VMEM((CAP + L,), jnp.int32),
            pltpu.VMEM((CAP + L,), jnp.int32),
            pltpu.VMEM((NT, CAP), jnp.int32),
            pltpu.VMEM((NT, CAP), jnp.int32),
            pltpu.VMEM((CAP + L,), jnp.int32),
            pltpu.VMEM((CAP + L,), jnp.int32),
            pltpu.VMEM((4, L), jnp.float32),
            pltpu.VMEM((L,), jnp.float32),
            pltpu.SMEM((A_TOT,), jnp.int32),
            pltpu.SemaphoreType.DMA,
        ],
    )(scores_pad, boxes_flat)
    return out.reshape(CAP, 5)[:K]


# speedup vs baseline: 1.0627x; 1.0627x over previous
import functools
import numpy as np
import jax
import jax.numpy as jnp
from jax import lax
from jax.experimental import pallas as pl
from jax.experimental.pallas import tpu as pltpu
from jax.experimental.pallas import tpu_sc as plsc

N = 20000
K = 1000
L = 16
NC = 2
NT = 16
NPAD = 20480
SH = NPAD // NT
SV = SH // L
CAP = 1024
SLOTS = CAP // (NC * NT)
GROUPS = SLOTS // L

NR = 32
A_CGT = NR
A_EVEC = NR + 1
A_CVEC = NR + 17
A_TOT = NR + 33

_MSK = np.int32(0x7FFFFFFF)
_ONE = np.int32(1)
_ZERO = np.int32(0)
_IMIN = np.int32(-0x80000000)


def _to_key(s):
    b = lax.bitcast_convert_type(s, jnp.int32)
    return jnp.where(b < 0, b ^ _MSK, b)


def _from_key(k):
    b = jnp.where(k < 0, k ^ _MSK, k)
    return lax.bitcast_convert_type(b, jnp.float32)


def _sc_body(scores_hbm, boxes_hbm, out_hbm, xk_hbm, xi_hbm,
             sv, kv, ck, ci, rk, ri, mk, mi, colb, stage5, smem,
             sem, bsem):
    core = lax.axis_index("c")
    w = lax.axis_index("s")
    ii = lax.iota(jnp.int32, L)

    def _z(i, _):
        smem[i] = _ZERO
        return 0
    lax.fori_loop(0, A_TOT, _z, 0)

    with jax.named_scope("p1_load"):
        @pl.when(w < NT - 1)
        def _():
            pltpu.sync_copy(scores_hbm.at[pl.ds(w * SH, SH)], sv)

        @pl.when(w == NT - 1)
        def _():
            pltpu.sync_copy(scores_hbm.at[pl.ds(w * SH, N - (NT - 1) * SH)],
                            sv.at[pl.ds(0, N - (NT - 1) * SH)])

        for i in range(SV):
            key = _to_key(sv[pl.ds(i * L, L)])
            if i * L >= N - (NT - 1) * SH:
                key = jnp.where(w == NT - 1, _IMIN + _ONE, key)
            kv[pl.ds(i * L, L)] = key

        plsc.subcore_barrier()

    def _count(pred):
        acc = jnp.zeros((L,), jnp.int32)
        for i in range(SV):
            v = kv[pl.ds(i * L, L)]
            acc = acc + jnp.where(pred(v), _ONE, _ZERO)
        return jnp.sum(acc)

    def _acc_read(word, local):
        plsc.fetch_and_add(smem.at[word], local, subcore_id=0)
        plsc.subcore_barrier()
        return plsc.fetch_and_add(smem.at[word], _ZERO, subcore_id=0)

    def _round(r, bstate):
        cand_b = bstate | (_ONE << (31 - r))
        cand = cand_b ^ _IMIN
        total = _acc_read(r, _count(lambda v: v >= cand))
        return jnp.where(total >= K, cand_b, bstate)

    with jax.named_scope("p2_search"):
        t_key = lax.fori_loop(0, NR, _round, _ZERO) ^ _IMIN

        c_gt = _acc_read(A_CGT, _count(lambda v: v > t_key))
    need_eq = np.int32(K) - c_gt

    with jax.named_scope("p3_quota"):
        e_local = _count(lambda v: v == t_key)
        plsc.fetch_and_add(smem.at[A_EVEC + w], e_local, subcore_id=0)
        plsc.subcore_barrier()

        def _pref(j, acc):
            return acc + plsc.fetch_and_add(smem.at[A_EVEC + j], _ZERO,
                                            subcore_id=0)
        e_before = lax.fori_loop(0, w, _pref, _ZERO)
        take_eq = jnp.clip(need_eq - e_before, _ZERO, e_local)

    def _comp(i, carry):
        off, eqs = carry
        v = kv[pl.ds(i * L, L)]
        m_gt = v > t_key
        m_eq = v == t_key
        eq_in = jnp.where(m_eq, _ONE, _ZERO)
        eq_rank = plsc.cumsum(eq_in) - eq_in + eqs
        sel = m_gt | (m_eq & (eq_rank < take_eq))
        idx = w * SH + i * L + ii
        sel_in = jnp.where(sel, _ONE, _ZERO)
        pos = off + plsc.cumsum(sel_in) - sel_in
        plsc.store_scatter(ck, [pos], v, mask=sel)
        plsc.store_scatter(ci, [pos], idx, mask=sel)
        return off + jnp.sum(sel_in), eqs + jnp.sum(eq_in)

    with jax.named_scope("p3_compact"):
        c_local, _ = lax.fori_loop(0, SV, _comp, (_ZERO, _ZERO))

    with jax.named_scope("p4_exchange"):
        row = core * NT + w
        pltpu.sync_copy(ck.at[pl.ds(0, CAP)], xk_hbm.at[row])
        pltpu.sync_copy(ci.at[pl.ds(0, CAP)], xi_hbm.at[row])
        plsc.fetch_and_add(smem.at[A_CVEC + w], c_local, subcore_id=0)
        plsc.subcore_barrier()

        pltpu.sync_copy(xk_hbm.at[pl.ds(core * NT, NT)], rk)
        pltpu.sync_copy(xi_hbm.at[pl.ds(core * NT, NT)], ri)

    def _cnt_of(j):
        return plsc.fetch_and_add(smem.at[A_CVEC + j], _ZERO, subcore_id=0)

    ns_merge = jax.named_scope("p4_merge")
    ns_merge.__enter__()
    pos_run = _ZERO
    for j in range(NT):
        cj = _cnt_of(j)
        pj = pos_run

        def _mrg(t, _, j=j, cj=cj, pj=pj):
            vv = rk[j, pl.ds(t * L, L)]
            iv = ri[j, pl.ds(t * L, L)]
            lpos = t * L + ii
            mm = lpos < cj
            plsc.store_scatter(mk, [pj + lpos], vv, mask=mm)
            plsc.store_scatter(mi, [pj + lpos], iv, mask=mm)
            return 0

        lax.fori_loop(0, (cj + (L - 1)) // L, _mrg, 0)
        pos_run = pos_run + cj

    for tpos in range((K // L), CAP // L):
        base = tpos * L
        gp = base + ii
        old_k = mk[pl.ds(base, L)]
        old_i = mi[pl.ds(base, L)]
        pad = gp >= K
        mk[pl.ds(base, L)] = jnp.where(pad, _IMIN, old_k)
        mi[pl.ds(base, L)] = jnp.where(pad, np.int32(N) + gp, old_i)
    ns_merge.__exit__(None, None, None)

    sks, sis = [], []
    for g in range(GROUPS):
        base = (core * NT + w) * SLOTS + g * L
        sks.append(mk[pl.ds(base, L)])
        sis.append(mi[pl.ds(base, L)])

    ranks = []
    with jax.named_scope("p5_rank"):
        for g in range(GROUPS):
            sk, si = sks[g], sis[g]

            def _rank(t, acc, sk=sk, si=si):
                ek = mk[pl.ds(t * L, L)]
                ei = mi[pl.ds(t * L, L)]
                for p in range(L):
                    perm = (ii + p) & (L - 1)
                    ekp = ek.at[perm].get(mode="promise_in_bounds")
                    eip = ei.at[perm].get(mode="promise_in_bounds")
                    beats = (ekp > sk) | ((ekp == sk) & (eip < si))
                    acc = acc + jnp.where(beats, _ONE, _ZERO)
                return acc

            ranks.append(
                lax.fori_loop(0, CAP // L, _rank, jnp.zeros((L,), jnp.int32)))

    with jax.named_scope("p6_scatter"):
        copies = []
        for g in range(GROUPS):
            gidx = jnp.where(sis[g] < N, sis[g], sis[g] - np.int32(N))
            for c in range(4):
                copies.append(
                    pltpu.async_copy(boxes_hbm.at[gidx * 4 + c],
                                     colb.at[g * 4 + c], bsem))
        outp = []
        for g in range(GROUPS):
            rank = ranks[g]
            stage5[pl.ds(g * L, L)] = _from_key(sks[g])
            for c in range(4):
                copies[g * 4 + c].wait()
                outp.append(
                    pltpu.async_copy(colb.at[g * 4 + c],
                                     out_hbm.at[rank * 5 + c], sem))
            outp.append(
                pltpu.async_copy(stage5.at[pl.ds(g * L, L)],
                                 out_hbm.at[rank * 5 + 4], sem))
        for d in outp:
            d.wait()


@jax.jit
def kernel(boxes, scores):
    mesh = plsc.VectorSubcoreMesh(
        core_axis_name="c", subcore_axis_name="s",
        num_cores=NC, num_subcores=NT)
    out, _, _ = pl.kernel(
        _sc_body,
        out_type=(
            jax.ShapeDtypeStruct((CAP * 5,), jnp.float32),
            jax.ShapeDtypeStruct((NC * NT, CAP), jnp.int32),
            jax.ShapeDtypeStruct((NC * NT, CAP), jnp.int32),
        ),
        mesh=mesh,
        compiler_params=pltpu.CompilerParams(needs_layout_passes=False),
        scratch_types=[
            pltpu.VMEM((SH,), jnp.float32),
            pltpu.VMEM((SH,), jnp.int32),
            pltpu.VMEM((CAP + L,), jnp.int32),
            pltpu.VMEM((CAP + L,), jnp.int32),
            pltpu.VMEM((NT, CAP), jnp.int32),
            pltpu.VMEM((NT, CAP), jnp.int32),
            pltpu.VMEM((CAP + L,), jnp.int32),
            pltpu.VMEM((CAP + L,), jnp.int32),
            pltpu.VMEM((4 * GROUPS, L), jnp.float32),
            pltpu.VMEM((GROUPS * L,), jnp.float32),
            pltpu.SMEM((A_TOT,), jnp.int32),
            pltpu.SemaphoreType.DMA,
            pltpu.SemaphoreType.DMA,
        ],
    )(scores, boxes.reshape(-1))
    return out.reshape(CAP, 5)[:K]
